# baseline (speedup 1.0000x reference)
import jax
import jax.numpy as jnp
from jax.experimental import pallas as pl

_BLOCK = 2000


def _distnet_kernel(x_ref, pts_ref, beta_ref, out_ref):
    i = pl.program_id(0)
    n = pl.num_programs(0)
    x = x_ref[...]
    pts = pts_ref[...]
    pts_t = pts.T
    pp = jnp.sum(pts_t * pts_t, axis=0, keepdims=True)
    lhs = jnp.concatenate(
        [jnp.ones((x.shape[0], 1), jnp.bfloat16), x.astype(jnp.bfloat16)],
        axis=1)
    rhs = jnp.concatenate([pp, -2.0 * pts_t], axis=0).astype(jnp.bfloat16)
    partial = jax.lax.dot_general(
        lhs, rhs, (((1,), (0,)), ((), ())),
        preferred_element_type=jnp.float32)
    mblk = jnp.min(partial, axis=1, keepdims=True)

    @pl.when(i == 0)
    def _():
        out_ref[...] = mblk

    @pl.when(i > 0)
    def _():
        out_ref[...] = jnp.minimum(out_ref[...], mblk)

    @pl.when(i == n - 1)
    def _():
        xx = jnp.sum(x * x, axis=1, keepdims=True)
        d = jnp.maximum(out_ref[...] + xx, 0.0)
        b = jax.nn.softplus(beta_ref[0, 0])
        alpha = -b * 6.9077542789816375
        out_ref[...] = jax.nn.sigmoid((d + alpha) / b)


def kernel(x, points, beta):
    q, dim = x.shape
    n_pts = points.shape[0]
    assert n_pts % _BLOCK == 0, n_pts
    n_blocks = n_pts // _BLOCK
    beta2d = beta.reshape(1, 1)
    out = pl.pallas_call(
        _distnet_kernel,
        grid=(n_blocks,),
        in_specs=[
            pl.BlockSpec((q, dim), lambda i: (0, 0)),
            pl.BlockSpec((_BLOCK, dim), lambda i: (i, 0)),
            pl.BlockSpec((1, 1), lambda i: (0, 0)),
        ],
        out_specs=pl.BlockSpec((q, 1), lambda i: (0, 0)),
        out_shape=jax.ShapeDtypeStruct((q, 1), jnp.float32),
    )(x, points, beta2d)
    return out.reshape(q)


# speedup vs baseline: 1.1080x; 1.1080x over previous
import jax
import jax.numpy as jnp
from jax.experimental import pallas as pl

_BLOCK = 4000
_SUB = 2000


def _distnet_kernel(x_ref, pts_ref, beta_ref, out_ref):
    i = pl.program_id(0)
    n = pl.num_programs(0)
    x = x_ref[...]
    pts = pts_ref[...]
    pts_t = pts.T
    pp = jnp.sum(pts_t * pts_t, axis=0, keepdims=True)
    lhs = jnp.concatenate(
        [jnp.ones((x.shape[0], 1), jnp.bfloat16), x.astype(jnp.bfloat16)],
        axis=1)
    rhs = jnp.concatenate([pp, -2.0 * pts_t], axis=0).astype(jnp.bfloat16)
    mblk = None
    for s in range(0, _BLOCK, _SUB):
        partial = jax.lax.dot_general(
            lhs, rhs[:, s:s + _SUB], (((1,), (0,)), ((), ())),
            preferred_element_type=jnp.float32)
        m = jnp.min(partial, axis=1, keepdims=True)
        mblk = m if mblk is None else jnp.minimum(mblk, m)

    @pl.when(i == 0)
    def _():
        out_ref[...] = mblk

    @pl.when(i > 0)
    def _():
        out_ref[...] = jnp.minimum(out_ref[...], mblk)

    @pl.when(i == n - 1)
    def _():
        xx = jnp.sum(x * x, axis=1, keepdims=True)
        d = jnp.maximum(out_ref[...] + xx, 0.0)
        b = jax.nn.softplus(beta_ref[0, 0])
        alpha = -b * 6.9077542789816375
        out_ref[...] = jax.nn.sigmoid((d + alpha) / b)


def kernel(x, points, beta):
    q, dim = x.shape
    n_pts = points.shape[0]
    assert n_pts % _BLOCK == 0, n_pts
    n_blocks = n_pts // _BLOCK
    beta2d = beta.reshape(1, 1)
    out = pl.pallas_call(
        _distnet_kernel,
        grid=(n_blocks,),
        in_specs=[
            pl.BlockSpec((q, dim), lambda i: (0, 0)),
            pl.BlockSpec((_BLOCK, dim), lambda i: (i, 0)),
            pl.BlockSpec((1, 1), lambda i: (0, 0)),
        ],
        out_specs=pl.BlockSpec((q, 1), lambda i: (0, 0)),
        out_shape=jax.ShapeDtypeStruct((q, 1), jnp.float32),
    )(x, points, beta2d)
    return out.reshape(q)


# speedup vs baseline: 1.1757x; 1.0611x over previous
import jax
import jax.numpy as jnp
from jax.experimental import pallas as pl

_BLOCK = 10000
_SUB = 2000


def _distnet_kernel(x_ref, pts_ref, beta_ref, out_ref):
    i = pl.program_id(0)
    n = pl.num_programs(0)
    x = x_ref[...]
    pts = pts_ref[...]
    pts_t = pts.T
    pp = jnp.sum(pts_t * pts_t, axis=0, keepdims=True)
    lhs = jnp.concatenate(
        [jnp.ones((x.shape[0], 1), jnp.bfloat16), x.astype(jnp.bfloat16)],
        axis=1)
    rhs = jnp.concatenate([pp, -2.0 * pts_t], axis=0).astype(jnp.bfloat16)
    mblk = None
    for s in range(0, _BLOCK, _SUB):
        partial = jax.lax.dot_general(
            lhs, rhs[:, s:s + _SUB], (((1,), (0,)), ((), ())),
            preferred_element_type=jnp.float32)
        m = jnp.min(partial, axis=1, keepdims=True)
        mblk = m if mblk is None else jnp.minimum(mblk, m)

    @pl.when(i == 0)
    def _():
        out_ref[...] = mblk

    @pl.when(i > 0)
    def _():
        out_ref[...] = jnp.minimum(out_ref[...], mblk)

    @pl.when(i == n - 1)
    def _():
        xx = jnp.sum(x * x, axis=1, keepdims=True)
        d = jnp.maximum(out_ref[...] + xx, 0.0)
        b = jax.nn.softplus(beta_ref[0, 0])
        alpha = -b * 6.9077542789816375
        out_ref[...] = jax.nn.sigmoid((d + alpha) / b)


def kernel(x, points, beta):
    q, dim = x.shape
    n_pts = points.shape[0]
    assert n_pts % _BLOCK == 0, n_pts
    n_blocks = n_pts // _BLOCK
    beta2d = beta.reshape(1, 1)
    out = pl.pallas_call(
        _distnet_kernel,
        grid=(n_blocks,),
        in_specs=[
            pl.BlockSpec((q, dim), lambda i: (0, 0)),
            pl.BlockSpec((_BLOCK, dim), lambda i: (i, 0)),
            pl.BlockSpec((1, 1), lambda i: (0, 0)),
        ],
        out_specs=pl.BlockSpec((q, 1), lambda i: (0, 0)),
        out_shape=jax.ShapeDtypeStruct((q, 1), jnp.float32),
    )(x, points, beta2d)
    return out.reshape(q)
